# Initial kernel scaffold; baseline (speedup 1.0000x reference)
import jax
import jax.numpy as jnp
from jax.experimental import pallas as pl


def kernel(feature_index, feature_value, label, feature_embedding, FM_first_order_weights, bias):
    raise NotImplementedError("write your pallas kernel here")


# speedup vs baseline: 1.3134x; 1.3134x over previous
import functools

import jax
import jax.numpy as jnp
from jax import lax
from jax.experimental import pallas as pl
from jax.experimental.pallas import tpu as pltpu
from jax.experimental.pallas import tpu_sc as plsc

F = 26
D = 26
N = 16384
NC = 2
NS = 16
NW = NC * NS
ROWS_PER_W = N // NW
CHUNK = 128
NCHUNK = ROWS_PER_W // CHUNK
IDX_PER_CHUNK = CHUNK * F
JSTREAMS = IDX_PER_CHUNK // 128


_GATHER_DNUMS = lax.GatherDimensionNumbers(
    offset_dims=(), collapsed_slice_dims=(0,), start_index_map=(0,))


def _bcast_lane(vec, lane):
    idx = jnp.full((16, 1), lane, jnp.int32)
    return lax.gather(vec, idx, _GATHER_DNUMS, slice_sizes=(1,),
                      mode=lax.GatherScatterMode.PROMISE_IN_BOUNDS)


def _fm_kernel(emb_hbm, w1_hbm, idx_hbm, val_hbm, bias_hbm, out_hbm,
               idx_v, val_v, rows_v, w1_v, out_v, bias_v, sem_e, sem_w):
    wid = lax.axis_index("s") * NC + lax.axis_index("c")

    pltpu.sync_copy(bias_hbm, bias_v)
    bias_vec = bias_v[...]

    def chunk_body(c, carry):
        gchunk = wid * NCHUNK + c
        vbase = gchunk * IDX_PER_CHUNK
        obase = gchunk * CHUNK

        pltpu.sync_copy(idx_hbm.at[pl.ds(vbase, IDX_PER_CHUNK)], idx_v)
        pltpu.sync_copy(val_hbm.at[pl.ds(vbase, IDX_PER_CHUNK)], val_v)

        def fire(j, carry2):
            isl = idx_v.at[pl.ds(j * 128, 128)]
            pltpu.async_copy(emb_hbm.at[isl],
                             rows_v.at[pl.ds(j * 128, 128), :], sem_e)
            pltpu.async_copy(w1_hbm.at[isl],
                             w1_v.at[pl.ds(j * 128, 128)], sem_w)
            return carry2

        lax.fori_loop(0, JSTREAMS, fire, 0)

        def drain(j, carry2):
            isl = idx_v.at[pl.ds(j * 128, 128)]
            pltpu.make_async_copy(emb_hbm.at[isl],
                                  rows_v.at[pl.ds(j * 128, 128), :],
                                  sem_e).wait()
            pltpu.make_async_copy(w1_hbm.at[isl],
                                  w1_v.at[pl.ds(j * 128, 128)],
                                  sem_w).wait()
            return carry2

        lax.fori_loop(0, JSTREAMS, drain, 0)

        def row_body(r, carry2):
            base = r * F
            zero = jnp.zeros((16,), jnp.float32)
            s_lo = zero
            s_hi = zero
            q_lo = zero
            q_hi = zero
            v_lo = val_v[pl.ds(base, 16)]
            v_hi = val_v[pl.ds(base + 10, 16)]
            for f in range(F):
                e_lo = rows_v[base + f, pl.ds(0, 16)]
                e_hi = rows_v[base + f, pl.ds(10, 16)]
                src, lane = (v_lo, f) if f < 16 else (v_hi, f - 10)
                vb = _bcast_lane(src, lane)
                ev_lo = e_lo * vb
                ev_hi = e_hi * vb
                s_lo = s_lo + ev_lo
                s_hi = s_hi + ev_hi
                q_lo = q_lo + ev_lo * ev_lo
                q_hi = q_hi + ev_hi * ev_hi
            yf_lo = w1_v[pl.ds(base, 16)] * v_lo
            yf_hi = w1_v[pl.ds(base + 10, 16)] * v_hi
            res_lo = yf_lo + 0.5 * (s_lo * s_lo - q_lo) + bias_vec
            res_hi = yf_hi + 0.5 * (s_hi * s_hi - q_hi) + bias_vec
            out_v[r, pl.ds(0, 16)] = res_lo
            out_v[r, pl.ds(10, 16)] = res_hi
            return carry2

        lax.fori_loop(0, CHUNK, row_body, 0)

        pltpu.sync_copy(out_v, out_hbm.at[pl.ds(obase, CHUNK), :])
        return carry

    lax.fori_loop(0, NCHUNK, chunk_body, 0)


@jax.jit
def _fm(feature_index, feature_value, feature_embedding, w1_flat, bias16):
    idx_flat = feature_index.reshape(-1)
    val_flat = feature_value.reshape(-1)
    mesh = plsc.VectorSubcoreMesh(
        core_axis_name="c", subcore_axis_name="s",
        num_cores=NC, num_subcores=NS)
    run = pl.kernel(
        _fm_kernel,
        out_type=jax.ShapeDtypeStruct((N, D), jnp.float32),
        mesh=mesh,
        scratch_types=[
            pltpu.VMEM((IDX_PER_CHUNK,), jnp.int32),
            pltpu.VMEM((IDX_PER_CHUNK,), jnp.float32),
            pltpu.VMEM((IDX_PER_CHUNK, D), jnp.float32),
            pltpu.VMEM((IDX_PER_CHUNK,), jnp.float32),
            pltpu.VMEM((CHUNK, D), jnp.float32),
            pltpu.VMEM((16,), jnp.float32),
            pltpu.SemaphoreType.DMA,
            pltpu.SemaphoreType.DMA,
        ],
        compiler_params=pltpu.CompilerParams(use_tc_tiling_on_sc=False),
    )
    return run(feature_embedding, w1_flat, idx_flat, val_flat, bias16)


def kernel(feature_index, feature_value, label, feature_embedding,
           FM_first_order_weights, bias):
    w1_flat = FM_first_order_weights.reshape(-1)
    bias16 = jnp.broadcast_to(bias, (16,))
    return _fm(feature_index, feature_value, feature_embedding,
               w1_flat, bias16)
